# baseline (speedup 1.0000x reference)
import functools

import jax
import jax.numpy as jnp
from jax import lax
from jax.experimental import pallas as pl
from jax.experimental.pallas import tpu as pltpu
from jax.experimental.pallas import tpu_sc as plsc

_H = 64
_NC = 2
_NS = 16
_NW = _NC * _NS
_CHUNK = 128
_KFIRE = 8


def _tc_head_table(t2, w2, b2):
    v2 = t2.shape[0]
    blk = 5000
    grid = (v2 // blk,)

    def body(t_ref, w_ref, b_ref, o_ref):
        s = jnp.tanh(0.5 * t_ref[...])
        o_ref[...] = (
            jnp.dot(s, w_ref[...], preferred_element_type=jnp.float32)
            + b_ref[...]
        )

    return pl.pallas_call(
        body,
        grid=grid,
        in_specs=[
            pl.BlockSpec((blk, 2 * _H), lambda i: (i, 0)),
            pl.BlockSpec((2 * _H, 4), lambda i: (0, 0)),
            pl.BlockSpec((1, 4), lambda i: (0, 0)),
        ],
        out_specs=pl.BlockSpec((blk, 4), lambda i: (i, 0)),
        out_shape=jax.ShapeDtypeStruct((v2, 4), jnp.float32),
    )(t2, w2, b2)


def _sc_lookup(flat, idx3):
    nw, n_chunks, chunk = idx3.shape
    n = nw * n_chunks * chunk
    n_super = n_chunks // _KFIRE
    sup = _KFIRE * chunk
    mesh = plsc.VectorSubcoreMesh(core_axis_name="c", subcore_axis_name="s")

    @functools.partial(
        pl.kernel,
        out_type=jax.ShapeDtypeStruct((n,), jnp.float32),
        mesh=mesh,
        compiler_params=pltpu.CompilerParams(use_tc_tiling_on_sc=False),
        scratch_types=[
            pltpu.VMEM((n_chunks, chunk), jnp.int32),
            pltpu.VMEM((sup,), jnp.float32),
            pltpu.SemaphoreType.DMA,
        ],
    )
    def k(flat_hbm, idx_hbm, out_hbm, idx_v, buf_v, gsem):
        wid = lax.axis_index("s") * _NC + lax.axis_index("c")
        pltpu.sync_copy(idx_hbm.at[wid], idx_v)

        def body(sb, carry):
            copies = []
            for bq in range(_KFIRE):
                j = sb * _KFIRE + bq
                copies.append(pltpu.async_copy(
                    flat_hbm.at[idx_v.at[j]],
                    buf_v.at[pl.ds(bq * chunk, chunk)], gsem))
            for c in copies:
                c.wait()
            base = (wid * n_super + sb) * sup
            pltpu.sync_copy(buf_v, out_hbm.at[pl.ds(base, sup)])
            return carry

        lax.fori_loop(0, n_super, body, 0)

    return k(flat, idx3)


def kernel(x, table, W, b):
    bsz, fields = x.shape
    v = table.shape[0]
    n = bsz * fields
    x2 = 2 * x.reshape(n, 1)
    xe = jnp.concatenate([x2, x2 + 1], axis=1)
    n_chunks = (2 * n) // (_NW * _CHUNK)
    idx3 = xe.reshape(_NW, n_chunks, _CHUNK)

    w2 = jnp.zeros((2 * _H, 4), jnp.float32)
    w2 = w2.at[:_H, :2].set(0.5 * W.T).at[_H:, 2:].set(0.5 * W.T)
    bj = b + 0.5 * W.sum(axis=1)
    b2 = jnp.concatenate([bj, bj]).reshape(1, 4)

    out4 = _tc_head_table(table.reshape(v // 2, 2 * _H), w2, b2)
    out = _sc_lookup(out4.reshape(2 * v), idx3)
    return out.reshape(bsz, fields, 2)
